# baseline (speedup 1.0000x reference)
import jax
import jax.numpy as jnp
from jax import lax
from jax.experimental import pallas as pl
from jax.experimental.pallas import tpu as pltpu
from jax.experimental.pallas import tpu_sc as plsc

SEQ = 200
BATCH = 4096
EMB = 64
VOCAB = 1000000
NC = 2
NS = 16
NW = NC * NS
BPW = BATCH // NW
LANES = 16
ESL = EMB // LANES

NTC_TOTAL = VOCAB // 128
TAIL_BASE = NTC_TOTAL * 128
TAIL = VOCAB - TAIL_BASE
NTC_BASE = NTC_TOTAL // NW
NTC_REM = NTC_TOTAL - NTC_BASE * NW
NBUF_A = 3
A_ITERS = (NTC_BASE + 1 + NBUF_A - 1) // NBUF_A


def _sc_transpose(in_t, out_flat, ib0, ib1, ib2,
                  pb0, pb1, pb2, tb,
                  is0, is1, is2, os0, os1, os2):
    c = lax.axis_index("c")
    s = lax.axis_index("s")
    w = c * NS + s
    ntc = NTC_BASE + jnp.where(w < NTC_REM, 1, 0)
    base = w * NTC_BASE + jnp.minimum(w, NTC_REM)

    ibufs = (ib0, ib1, ib2)
    pbufs = (pb0, pb1, pb2)
    isems = (is0, is1, is2)
    osems = (os0, os1, os2)
    iota = lax.broadcasted_iota(jnp.int32, (LANES,), 0)

    def start_in(i, b):
        tc = base + i
        pltpu.async_copy(in_t.at[:, pl.ds(tc * 128, 128)], ibufs[b], isems[b])

    def wait_in(b):
        pltpu.make_async_copy(in_t.at[:, pl.ds(0, 128)], ibufs[b],
                              isems[b]).wait()

    def start_out(i, b):
        tc = base + i
        pltpu.async_copy(pbufs[b], out_flat.at[pl.ds(tc * 4096, 4096)],
                         osems[b])

    def wait_out(b):
        pltpu.make_async_copy(pbufs[b], out_flat.at[pl.ds(0, 4096)],
                              osems[b]).wait()

    rows4 = [eg * LANES + iota for eg in range(ESL)]
    zvec = jnp.zeros((LANES,), jnp.int32)

    def transpose_buf(b):
        inb = ibufs[b]
        pb = pbufs[b]

        @plsc.parallel_loop(0, 128, step=1, unroll=8, carry=zvec)
        def _(l, lvec):
            lmod = (lvec + iota) & 127
            lm32 = lmod * (EMB // 2)
            for h in range(EMB // 32):
                v0 = plsc.load_gather(inb, [rows4[2 * h], lmod])
                v1 = plsc.load_gather(inb, [rows4[2 * h + 1], lmod])
                u = (plsc.bitcast(v0, jnp.uint32) & jnp.uint32(0xFFFF0000)) | (
                    plsc.bitcast(v1, jnp.uint32) >> 16)
                plsc.store_scatter(pb, [lm32 + h * LANES + iota],
                                   plsc.bitcast(u, jnp.float32))
            return lvec + 1

    for b in range(NBUF_A):
        @pl.when(b < ntc)
        def _():
            start_in(b, b)

    def body(g, carry):
        for b in range(NBUF_A):
            i = g * NBUF_A + b

            @pl.when(i < ntc)
            def _():
                wait_in(b)

                @pl.when(i >= NBUF_A)
                def _():
                    wait_out(b)

                transpose_buf(b)
                start_out(i, b)

                @pl.when(i + NBUF_A < ntc)
                def _():
                    start_in(i + NBUF_A, b)
        return carry

    lax.fori_loop(0, A_ITERS, body, 0)

    for b in range(NBUF_A):
        @pl.when(ntc > b)
        def _():
            wait_out(b)

    @pl.when(w == NW - 1)
    def _():
        pltpu.sync_copy(in_t.at[:, pl.ds(TAIL_BASE, TAIL)], tb)

        @plsc.parallel_loop(0, TAIL, step=1, unroll=8, carry=zvec)
        def _(l, lvec):
            lmod = (lvec + iota) & (TAIL - 1)
            lm32 = lmod * (EMB // 2)
            for h in range(EMB // 32):
                v0 = plsc.load_gather(tb, [rows4[2 * h], lmod])
                v1 = plsc.load_gather(tb, [rows4[2 * h + 1], lmod])
                u = (plsc.bitcast(v0, jnp.uint32) & jnp.uint32(0xFFFF0000)) | (
                    plsc.bitcast(v1, jnp.uint32) >> 16)
                plsc.store_scatter(pb0, [lm32 + h * LANES + iota],
                                   plsc.bitcast(u, jnp.float32))
            return lvec + 1
        pltpu.sync_copy(pb0.at[pl.ds(0, TAIL * EMB // 2)],
                        out_flat.at[pl.ds(TAIL_BASE * EMB // 2,
                                          TAIL * EMB // 2)])


def _linearize_table(emb_table):
    mesh = plsc.VectorSubcoreMesh(core_axis_name="c", subcore_axis_name="s")
    fn = pl.kernel(
        _sc_transpose,
        out_type=jax.ShapeDtypeStruct((VOCAB * EMB // 2,), jnp.float32),
        mesh=mesh,
        scratch_types=[
            pltpu.VMEM((EMB, 128), jnp.float32),
            pltpu.VMEM((EMB, 128), jnp.float32),
            pltpu.VMEM((EMB, 128), jnp.float32),
            pltpu.VMEM((128 * EMB // 2,), jnp.float32),
            pltpu.VMEM((128 * EMB // 2,), jnp.float32),
            pltpu.VMEM((128 * EMB // 2,), jnp.float32),
            pltpu.VMEM((EMB, TAIL), jnp.float32),
            pltpu.SemaphoreType.DMA,
            pltpu.SemaphoreType.DMA,
            pltpu.SemaphoreType.DMA,
            pltpu.SemaphoreType.DMA,
            pltpu.SemaphoreType.DMA,
            pltpu.SemaphoreType.DMA,
        ],
        compiler_params=pltpu.CompilerParams(needs_layout_passes=False),
    )
    return fn(emb_table.T)


def _sc_pool_sum(docs_hbm, emb_hbm, out_hbm, idx_v, buf0, buf1, buf2, buf3,
                 acc, sem0, sem1, sem2, sem3):
    c = lax.axis_index("c")
    s = lax.axis_index("s")
    wid = c * NS + s
    base = wid * BPW

    pltpu.sync_copy(docs_hbm.at[:, pl.ds(base, BPW)], idx_v)

    zeros = jnp.zeros((LANES,), jnp.float32)

    def zero_row(r, carry):
        for e in range(ESL):
            acc[r, pl.ds(e * LANES, LANES)] = zeros
        return carry

    lax.fori_loop(0, BPW, zero_row, 0, unroll=4)

    bufs = (buf0, buf1, buf2, buf3)
    sems = (sem0, sem1, sem2, sem3)

    def start(step, b):
        pltpu.async_copy(emb_hbm.at[idx_v.at[step]], bufs[b], sems[b])

    def wait(b):
        pltpu.make_async_copy(emb_hbm.at[idx_v.at[0]], bufs[b], sems[b]).wait()

    def accum(b):
        buf = bufs[b]

        def row(r, carry):
            for h in range(EMB // 32):
                x = buf[r, pl.ds(h * LANES, LANES)]
                u = plsc.bitcast(x, jnp.uint32)
                a = plsc.bitcast(u & jnp.uint32(0xFFFF0000), jnp.float32)
                b2 = plsc.bitcast(u << 16, jnp.float32)
                plsc.addupdate(acc.at[r, pl.ds(h * 32, LANES)], a)
                plsc.addupdate(acc.at[r, pl.ds(h * 32 + LANES, LANES)], b2)
            return carry

        lax.fori_loop(0, BPW, row, 0, unroll=4)

    for b in range(4):
        start(b, b)

    def body(g, carry):
        for b in range(4):
            step = 4 * g + b
            wait(b)
            accum(b)

            @pl.when(step + 4 < SEQ)
            def _():
                start(step + 4, b)
        return carry

    lax.fori_loop(0, SEQ // 4, body, 0)

    pltpu.sync_copy(acc, out_hbm.at[pl.ds(base, BPW)])


def _pool_sum_sc(docs, emb_linear):
    mesh = plsc.VectorSubcoreMesh(core_axis_name="c", subcore_axis_name="s")
    fn = pl.kernel(
        _sc_pool_sum,
        out_type=jax.ShapeDtypeStruct((BATCH, EMB), jnp.float32),
        mesh=mesh,
        scratch_types=[
            pltpu.VMEM((SEQ, BPW), jnp.int32),
            pltpu.VMEM((BPW, EMB // 2), jnp.float32),
            pltpu.VMEM((BPW, EMB // 2), jnp.float32),
            pltpu.VMEM((BPW, EMB // 2), jnp.float32),
            pltpu.VMEM((BPW, EMB // 2), jnp.float32),
            pltpu.VMEM((BPW, EMB), jnp.float32),
            pltpu.SemaphoreType.DMA,
            pltpu.SemaphoreType.DMA,
            pltpu.SemaphoreType.DMA,
            pltpu.SemaphoreType.DMA,
        ],
        compiler_params=pltpu.CompilerParams(use_tc_tiling_on_sc=False,
                                             needs_layout_passes=False),
    )
    return fn(docs, emb_linear)


def _mlp_body(pool_ref, w1_ref, b1_ref, w2_ref, b2_ref, out_ref):
    x = pool_ref[...] * (1.0 / SEQ)
    h = lax.dot_general(x, w1_ref[...], (((1,), (1,)), ((), ())),
                        preferred_element_type=jnp.float32)
    h = jnp.maximum(h + b1_ref[...], 0.0)
    o = lax.dot_general(h, w2_ref[...], (((1,), (1,)), ((), ())),
                        preferred_element_type=jnp.float32)
    out_ref[...] = o + b2_ref[...]


def _mlp_tc(pool_sum, W1, b1, W2, b2):
    return pl.pallas_call(
        _mlp_body,
        out_shape=jax.ShapeDtypeStruct((BATCH, W2.shape[0]), jnp.float32),
    )(pool_sum, W1, b1.reshape(1, -1), W2, b2.reshape(1, -1))


@jax.jit
def kernel(docs, emb_table, W1, b1, W2, b2):
    emb_linear = _linearize_table(emb_table).reshape(VOCAB, EMB // 2)
    pool_sum = _pool_sum_sc(docs, emb_linear)
    return _mlp_tc(pool_sum, W1, b1, W2, b2)


# speedup vs baseline: 1.5365x; 1.5365x over previous
import jax
import jax.numpy as jnp
from jax import lax
from jax.experimental import pallas as pl
from jax.experimental.pallas import tpu as pltpu
from jax.experimental.pallas import tpu_sc as plsc

SEQ = 200
BATCH = 4096
EMB = 64
VOCAB = 1000000
NC = 2
NS = 16
NW = NC * NS
BPW = BATCH // NW
LANES = 16
ESL = EMB // LANES

NTC_TOTAL = VOCAB // 128
TAIL_BASE = NTC_TOTAL * 128
TAIL = VOCAB - TAIL_BASE
NTC_BASE = NTC_TOTAL // NW
NTC_REM = NTC_TOTAL - NTC_BASE * NW
NBUF_A = 3
A_ITERS = (NTC_BASE + 1 + NBUF_A - 1) // NBUF_A


def _sc_transpose(in_t, out_flat, ib0, ib1, ib2,
                  pb0, pb1, pb2, tb,
                  is0, is1, is2, os0, os1, os2):
    c = lax.axis_index("c")
    s = lax.axis_index("s")
    w = c * NS + s
    ntc = NTC_BASE + jnp.where(w < NTC_REM, 1, 0)
    base = w * NTC_BASE + jnp.minimum(w, NTC_REM)

    ibufs = (ib0, ib1, ib2)
    pbufs = (pb0, pb1, pb2)
    isems = (is0, is1, is2)
    osems = (os0, os1, os2)
    iota = lax.broadcasted_iota(jnp.int32, (LANES,), 0)

    def start_in(i, b):
        tc = base + i
        pltpu.async_copy(in_t.at[:, pl.ds(tc * 128, 128)], ibufs[b], isems[b])

    def wait_in(b):
        pltpu.make_async_copy(in_t.at[:, pl.ds(0, 128)], ibufs[b],
                              isems[b]).wait()

    def start_out(i, b):
        tc = base + i
        pltpu.async_copy(pbufs[b], out_flat.at[pl.ds(tc * 4096, 4096)],
                         osems[b])

    def wait_out(b):
        pltpu.make_async_copy(pbufs[b], out_flat.at[pl.ds(0, 4096)],
                              osems[b]).wait()

    rows4 = [eg * LANES + iota for eg in range(ESL)]
    zvec = jnp.zeros((LANES,), jnp.int32)

    def transpose_buf(b):
        inb = ibufs[b]
        pb = pbufs[b]

        @plsc.parallel_loop(0, 128, step=1, unroll=8, carry=zvec)
        def _(l, lvec):
            lmod = (lvec + iota) & 127
            lm32 = lmod * (EMB // 2)
            for h in range(EMB // 32):
                v0 = plsc.load_gather(inb, [rows4[2 * h], lmod])
                v1 = plsc.load_gather(inb, [rows4[2 * h + 1], lmod])
                u = (plsc.bitcast(v0, jnp.uint32) & jnp.uint32(0xFFFF0000)) | (
                    plsc.bitcast(v1, jnp.uint32) >> 16)
                plsc.store_scatter(pb, [lm32 + h * LANES + iota],
                                   plsc.bitcast(u, jnp.float32))
            return lvec + 1

    for b in range(NBUF_A):
        @pl.when(b < ntc)
        def _():
            start_in(b, b)

    def body(g, carry):
        for b in range(NBUF_A):
            i = g * NBUF_A + b

            @pl.when(i < ntc)
            def _():
                wait_in(b)

                @pl.when(i >= NBUF_A)
                def _():
                    wait_out(b)

                transpose_buf(b)
                start_out(i, b)

                @pl.when(i + NBUF_A < ntc)
                def _():
                    start_in(i + NBUF_A, b)
        return carry

    lax.fori_loop(0, A_ITERS, body, 0)

    for b in range(NBUF_A):
        @pl.when(ntc > b)
        def _():
            wait_out(b)

    @pl.when(w == NW - 1)
    def _():
        pltpu.sync_copy(in_t.at[:, pl.ds(TAIL_BASE, TAIL)], tb)

        @plsc.parallel_loop(0, TAIL, step=1, unroll=8, carry=zvec)
        def _(l, lvec):
            lmod = (lvec + iota) & (TAIL - 1)
            lm32 = lmod * (EMB // 2)
            for h in range(EMB // 32):
                v0 = plsc.load_gather(tb, [rows4[2 * h], lmod])
                v1 = plsc.load_gather(tb, [rows4[2 * h + 1], lmod])
                u = (plsc.bitcast(v0, jnp.uint32) & jnp.uint32(0xFFFF0000)) | (
                    plsc.bitcast(v1, jnp.uint32) >> 16)
                plsc.store_scatter(pb0, [lm32 + h * LANES + iota],
                                   plsc.bitcast(u, jnp.float32))
            return lvec + 1
        pltpu.sync_copy(pb0.at[pl.ds(0, TAIL * EMB // 2)],
                        out_flat.at[pl.ds(TAIL_BASE * EMB // 2,
                                          TAIL * EMB // 2)])


def _linearize_table(emb_table):
    mesh = plsc.VectorSubcoreMesh(core_axis_name="c", subcore_axis_name="s")
    fn = pl.kernel(
        _sc_transpose,
        out_type=jax.ShapeDtypeStruct((VOCAB * EMB // 2,), jnp.float32),
        mesh=mesh,
        scratch_types=[
            pltpu.VMEM((EMB, 128), jnp.float32),
            pltpu.VMEM((EMB, 128), jnp.float32),
            pltpu.VMEM((EMB, 128), jnp.float32),
            pltpu.VMEM((128 * EMB // 2,), jnp.float32),
            pltpu.VMEM((128 * EMB // 2,), jnp.float32),
            pltpu.VMEM((128 * EMB // 2,), jnp.float32),
            pltpu.VMEM((EMB, TAIL), jnp.float32),
            pltpu.SemaphoreType.DMA,
            pltpu.SemaphoreType.DMA,
            pltpu.SemaphoreType.DMA,
            pltpu.SemaphoreType.DMA,
            pltpu.SemaphoreType.DMA,
            pltpu.SemaphoreType.DMA,
        ],
        compiler_params=pltpu.CompilerParams(needs_layout_passes=False),
    )
    return fn(emb_table.T)


def _sc_pool_sum(docs_hbm, emb_hbm, out_hbm, idx_v, buf0, buf1, buf2, buf3,
                 acc, sem0, sem1, sem2, sem3):
    c = lax.axis_index("c")
    s = lax.axis_index("s")
    wid = c * NS + s
    base = wid * BPW

    pltpu.sync_copy(docs_hbm.at[:, pl.ds(base, BPW)], idx_v)

    zeros = jnp.zeros((LANES,), jnp.float32)

    def zero_row(r, carry):
        for e in range(ESL):
            acc[r, pl.ds(e * LANES, LANES)] = zeros
        return carry

    lax.fori_loop(0, BPW, zero_row, 0, unroll=4)

    bufs = (buf0, buf1, buf2, buf3)
    sems = (sem0, sem1, sem2, sem3)

    def start(step, b):
        pltpu.async_copy(emb_hbm.at[idx_v.at[step]], bufs[b], sems[b])

    def wait(b):
        pltpu.make_async_copy(emb_hbm.at[idx_v.at[0]], bufs[b], sems[b]).wait()

    def accum(b):
        buf = bufs[b]

        @plsc.parallel_loop(0, BPW, step=1, unroll=8)
        def _(r):
            for h in range(EMB // 32):
                x = buf[r, pl.ds(h * LANES, LANES)]
                u = plsc.bitcast(x, jnp.uint32)
                a = plsc.bitcast(u & jnp.uint32(0xFFFF0000), jnp.float32)
                b2 = plsc.bitcast(u << 16, jnp.float32)
                plsc.addupdate(acc.at[r, pl.ds(h * 32, LANES)], a)
                plsc.addupdate(acc.at[r, pl.ds(h * 32 + LANES, LANES)], b2)

    for b in range(4):
        start(b, b)

    def body(g, carry):
        for b in range(4):
            step = 4 * g + b
            wait(b)
            accum(b)

            @pl.when(step + 4 < SEQ)
            def _():
                start(step + 4, b)
        return carry

    lax.fori_loop(0, SEQ // 4, body, 0)

    pltpu.sync_copy(acc, out_hbm.at[pl.ds(base, BPW)])


def _pool_sum_sc(docs, emb_linear):
    mesh = plsc.VectorSubcoreMesh(core_axis_name="c", subcore_axis_name="s")
    fn = pl.kernel(
        _sc_pool_sum,
        out_type=jax.ShapeDtypeStruct((BATCH, EMB), jnp.float32),
        mesh=mesh,
        scratch_types=[
            pltpu.VMEM((SEQ, BPW), jnp.int32),
            pltpu.VMEM((BPW, EMB // 2), jnp.float32),
            pltpu.VMEM((BPW, EMB // 2), jnp.float32),
            pltpu.VMEM((BPW, EMB // 2), jnp.float32),
            pltpu.VMEM((BPW, EMB // 2), jnp.float32),
            pltpu.VMEM((BPW, EMB), jnp.float32),
            pltpu.SemaphoreType.DMA,
            pltpu.SemaphoreType.DMA,
            pltpu.SemaphoreType.DMA,
            pltpu.SemaphoreType.DMA,
        ],
        compiler_params=pltpu.CompilerParams(use_tc_tiling_on_sc=False,
                                             needs_layout_passes=False),
    )
    return fn(docs, emb_linear)


def _mlp_body(pool_ref, w1_ref, b1_ref, w2_ref, b2_ref, out_ref):
    x = pool_ref[...] * (1.0 / SEQ)
    h = lax.dot_general(x, w1_ref[...], (((1,), (1,)), ((), ())),
                        preferred_element_type=jnp.float32)
    h = jnp.maximum(h + b1_ref[...], 0.0)
    o = lax.dot_general(h, w2_ref[...], (((1,), (1,)), ((), ())),
                        preferred_element_type=jnp.float32)
    out_ref[...] = o + b2_ref[...]


def _mlp_tc(pool_sum, W1, b1, W2, b2):
    return pl.pallas_call(
        _mlp_body,
        out_shape=jax.ShapeDtypeStruct((BATCH, W2.shape[0]), jnp.float32),
    )(pool_sum, W1, b1.reshape(1, -1), W2, b2.reshape(1, -1))


@jax.jit
def kernel(docs, emb_table, W1, b1, W2, b2):
    emb_linear = _linearize_table(emb_table).reshape(VOCAB, EMB // 2)
    pool_sum = _pool_sum_sc(docs, emb_linear)
    return _mlp_tc(pool_sum, W1, b1, W2, b2)
